# Initial kernel scaffold; baseline (speedup 1.0000x reference)
import jax
import jax.numpy as jnp
from jax.experimental import pallas as pl


def kernel(x):
    raise NotImplementedError("write your pallas kernel here")


# speedup vs baseline: 1.2585x; 1.2585x over previous
import functools

import numpy as np

import jax
import jax.numpy as jnp
from jax.experimental import pallas as pl
from jax.experimental.pallas import tpu as pltpu


def _interp_taps(in_size: int, out_size: int) -> np.ndarray:
    scale = np.float32(in_size / out_size)
    src = (np.arange(out_size, dtype=np.float32) + np.float32(0.5)) * scale - np.float32(0.5)
    src = np.maximum(src, np.float32(0.0))
    lo = np.minimum(np.floor(src).astype(np.int64), in_size - 1)
    hi = np.minimum(lo + 1, in_size - 1)
    frac = (src - lo.astype(np.float32)).astype(np.float32)
    mat = np.zeros((out_size, in_size), np.float32)
    np.add.at(mat, (np.arange(out_size), lo), np.float32(1.0) - frac)
    np.add.at(mat, (np.arange(out_size), hi), frac)
    return mat


def _kron_weight(h: int, w: int, h_out: int, w_out: int) -> np.ndarray:
    wh_t = _interp_taps(h, h_out).T
    ww_t = _interp_taps(w, w_out).T
    k = wh_t[:, None, :, None] * ww_t[None, :, None, :]
    return np.ascontiguousarray(k.reshape(h * w, h_out * w_out))


def _block_matmul(w_ref, x_ref, o_ref):
    o_ref[...] = jax.lax.dot_general(
        x_ref[...], w_ref[...],
        dimension_numbers=(((1,), (0,)), ((), ())),
        preferred_element_type=jnp.float32,
    ).astype(o_ref.dtype)


def _pick_block(nc: int, row_in: int, row_out: int) -> int:
    budget = 20 * 1024 * 1024
    per_row = 2 * 4 * (row_in + row_out)
    cap = max(8, budget // per_row)
    best = 8
    for b in range(8, min(nc, cap) + 1, 8):
        if nc % b == 0:
            best = b
    return best


@functools.partial(jax.jit, static_argnames=("scale",))
def _upsample2d(x: jnp.ndarray, scale: int) -> jnp.ndarray:
    n, c, h, w = x.shape
    h_out, w_out = h * scale, w * scale
    nc = n * c
    row_in, row_out = h * w, h_out * w_out

    k_mat = jnp.asarray(_kron_weight(h, w, h_out, w_out))
    x2d = x.reshape(nc, row_in)

    b = _pick_block(nc, row_in, row_out)
    pad = (-nc) % b
    if pad:
        x2d = jnp.pad(x2d, ((0, pad), (0, 0)))
    rows = nc + pad

    out2d = pl.pallas_call(
        _block_matmul,
        out_shape=jax.ShapeDtypeStruct((rows, row_out), x.dtype),
        grid=(rows // b,),
        in_specs=[
            pl.BlockSpec((row_in, row_out), lambda i: (0, 0)),
            pl.BlockSpec((b, row_in), lambda i: (i, 0)),
        ],
        out_specs=pl.BlockSpec((b, row_out), lambda i: (i, 0)),
        compiler_params=pltpu.CompilerParams(
            dimension_semantics=("parallel",),
            vmem_limit_bytes=64 * 1024 * 1024,
        ),
        cost_estimate=pl.CostEstimate(
            flops=2 * rows * row_in * row_out,
            transcendentals=0,
            bytes_accessed=4 * (rows * (row_in + row_out) + row_in * row_out),
        ),
    )(k_mat, x2d)

    if pad:
        out2d = out2d[:nc]
    return out2d.reshape(n, c, h_out, w_out)


def kernel(x):
    return _upsample2d(x, scale=2)


# speedup vs baseline: 5.4774x; 4.3524x over previous
import functools

import numpy as np

import jax
import jax.numpy as jnp
from jax.experimental import pallas as pl
from jax.experimental.pallas import tpu as pltpu


def _interp_taps(in_size: int, out_size: int) -> np.ndarray:
    scale = np.float32(in_size / out_size)
    src = (np.arange(out_size, dtype=np.float32) + np.float32(0.5)) * scale - np.float32(0.5)
    src = np.maximum(src, np.float32(0.0))
    lo = np.minimum(np.floor(src).astype(np.int64), in_size - 1)
    hi = np.minimum(lo + 1, in_size - 1)
    frac = (src - lo.astype(np.float32)).astype(np.float32)
    mat = np.zeros((out_size, in_size), np.float32)
    np.add.at(mat, (np.arange(out_size), lo), np.float32(1.0) - frac)
    np.add.at(mat, (np.arange(out_size), hi), frac)
    return mat


def _left_kron(h: int, w: int, h_out: int, w_out: int) -> np.ndarray:
    return np.kron(_interp_taps(h, h_out), _interp_taps(w, w_out))


def _upsample_block(k_ref, x_ref, o_ref):
    b, h, w, c = x_ref.shape
    hw_out = k_ref.shape[0]
    x_slabs = x_ref[...].reshape(b, h * w, c)
    out = jax.lax.dot_general(
        k_ref[...], x_slabs,
        dimension_numbers=(((1,), (1,)), ((), ())),
        preferred_element_type=jnp.float32,
    )
    o_ref[...] = jnp.swapaxes(out, 0, 1).reshape(o_ref.shape).astype(o_ref.dtype)


def _upsample_block1(k_ref, x_ref, o_ref):
    _, h, w, c = x_ref.shape
    o_ref[...] = jax.lax.dot_general(
        k_ref[...], x_ref[...].reshape(h * w, c),
        dimension_numbers=(((1,), (0,)), ((), ())),
        preferred_element_type=jnp.float32,
    ).reshape(o_ref.shape).astype(o_ref.dtype)


@functools.partial(jax.jit, static_argnames=("scale",))
def _upsample_nhwc(x: jnp.ndarray, scale: int) -> jnp.ndarray:
    n, c, h, w = x.shape
    h_out, w_out = h * scale, w * scale

    xt = jnp.transpose(x, (0, 2, 3, 1))
    k_mat = jnp.asarray(_left_kron(h, w, h_out, w_out))

    out_t = pl.pallas_call(
        _upsample_block1,
        out_shape=jax.ShapeDtypeStruct((n, h_out, w_out, c), x.dtype),
        grid=(n,),
        in_specs=[
            pl.BlockSpec((h_out * w_out, h * w), lambda i: (0, 0)),
            pl.BlockSpec((1, h, w, c), lambda i: (i, 0, 0, 0)),
        ],
        out_specs=pl.BlockSpec((1, h_out, w_out, c), lambda i: (i, 0, 0, 0)),
        compiler_params=pltpu.CompilerParams(
            dimension_semantics=("parallel",),
            vmem_limit_bytes=64 * 1024 * 1024,
        ),
        cost_estimate=pl.CostEstimate(
            flops=2 * n * c * h * w * h_out * w_out,
            transcendentals=0,
            bytes_accessed=4 * (n * c * (h * w + h_out * w_out) + h * w * h_out * w_out),
        ),
    )(k_mat, xt)

    return jnp.transpose(out_t, (0, 3, 1, 2))


def kernel(x):
    return _upsample_nhwc(x, scale=2)


# speedup vs baseline: 10.1198x; 1.8476x over previous
import functools

import numpy as np

import jax
import jax.numpy as jnp
from jax.experimental import pallas as pl
from jax.experimental.pallas import tpu as pltpu


def _interp_taps(in_size: int, out_size: int) -> np.ndarray:
    scale = np.float32(in_size / out_size)
    src = (np.arange(out_size, dtype=np.float32) + np.float32(0.5)) * scale - np.float32(0.5)
    src = np.maximum(src, np.float32(0.0))
    lo = np.minimum(np.floor(src).astype(np.int64), in_size - 1)
    hi = np.minimum(lo + 1, in_size - 1)
    frac = (src - lo.astype(np.float32)).astype(np.float32)
    mat = np.zeros((out_size, in_size), np.float32)
    np.add.at(mat, (np.arange(out_size), lo), np.float32(1.0) - frac)
    np.add.at(mat, (np.arange(out_size), hi), frac)
    return mat


def _left_kron(h: int, w: int, h_out: int, w_out: int) -> np.ndarray:
    return np.kron(_interp_taps(h, h_out), _interp_taps(w, w_out))


def _upsample_block(k_ref, x_ref, o_ref):
    b, h, w, c = x_ref.shape
    ho, wo = o_ref.shape[1], o_ref.shape[2]
    for j in range(b):
        o_ref[j] = jax.lax.dot_general(
            k_ref[...], x_ref[j].reshape(h * w, c),
            dimension_numbers=(((1,), (0,)), ((), ())),
            preferred_element_type=jnp.float32,
        ).reshape(ho, wo, c).astype(o_ref.dtype)


@functools.partial(jax.jit, static_argnames=("scale",))
def _upsample_nhwc(x: jnp.ndarray, scale: int) -> jnp.ndarray:
    n, c, h, w = x.shape
    h_out, w_out = h * scale, w * scale

    xt = jnp.transpose(x, (0, 2, 3, 1))
    k_mat = jnp.asarray(_left_kron(h, w, h_out, w_out))

    blk = 4
    while n % blk:
        blk //= 2

    out_t = pl.pallas_call(
        _upsample_block,
        out_shape=jax.ShapeDtypeStruct((n, h_out, w_out, c), x.dtype),
        grid=(n // blk,),
        in_specs=[
            pl.BlockSpec((h_out * w_out, h * w), lambda i: (0, 0)),
            pl.BlockSpec((blk, h, w, c), lambda i: (i, 0, 0, 0)),
        ],
        out_specs=pl.BlockSpec((blk, h_out, w_out, c), lambda i: (i, 0, 0, 0)),
        compiler_params=pltpu.CompilerParams(
            dimension_semantics=("parallel",),
            vmem_limit_bytes=64 * 1024 * 1024,
        ),
        cost_estimate=pl.CostEstimate(
            flops=2 * n * c * h * w * h_out * w_out,
            transcendentals=0,
            bytes_accessed=4 * (n * c * (h * w + h_out * w_out) + h * w * h_out * w_out),
        ),
    )(k_mat, xt)

    return jnp.transpose(out_t, (0, 3, 1, 2))


def kernel(x):
    return _upsample_nhwc(x, scale=2)


# speedup vs baseline: 11.2327x; 1.1100x over previous
import functools

import numpy as np

import jax
import jax.numpy as jnp
from jax.experimental import pallas as pl
from jax.experimental.pallas import tpu as pltpu


def _interp_taps(in_size: int, out_size: int) -> np.ndarray:
    scale = np.float32(in_size / out_size)
    src = (np.arange(out_size, dtype=np.float32) + np.float32(0.5)) * scale - np.float32(0.5)
    src = np.maximum(src, np.float32(0.0))
    lo = np.minimum(np.floor(src).astype(np.int64), in_size - 1)
    hi = np.minimum(lo + 1, in_size - 1)
    frac = (src - lo.astype(np.float32)).astype(np.float32)
    mat = np.zeros((out_size, in_size), np.float32)
    np.add.at(mat, (np.arange(out_size), lo), np.float32(1.0) - frac)
    np.add.at(mat, (np.arange(out_size), hi), frac)
    return mat


def _left_kron(h: int, w: int, h_out: int, w_out: int) -> np.ndarray:
    return np.kron(_interp_taps(h, h_out), _interp_taps(w, w_out))


def _upsample_block(k_ref, x_ref, o_ref):
    b, h, w, c = x_ref.shape
    ho, wo = o_ref.shape[1], o_ref.shape[2]
    for j in range(b):
        o_ref[j] = jax.lax.dot_general(
            k_ref[...], x_ref[j].reshape(h * w, c),
            dimension_numbers=(((1,), (0,)), ((), ())),
            preferred_element_type=jnp.float32,
        ).reshape(ho, wo, c).astype(o_ref.dtype)


@functools.partial(jax.jit, static_argnames=("scale",))
def _upsample_nhwc(x: jnp.ndarray, scale: int) -> jnp.ndarray:
    n, c, h, w = x.shape
    h_out, w_out = h * scale, w * scale

    xt = jnp.transpose(x, (0, 2, 3, 1))
    k_mat = jnp.asarray(_left_kron(h, w, h_out, w_out))

    blk = 8
    while n % blk:
        blk //= 2

    out_t = pl.pallas_call(
        _upsample_block,
        out_shape=jax.ShapeDtypeStruct((n, h_out, w_out, c), x.dtype),
        grid=(n // blk,),
        in_specs=[
            pl.BlockSpec((h_out * w_out, h * w), lambda i: (0, 0)),
            pl.BlockSpec((blk, h, w, c), lambda i: (i, 0, 0, 0)),
        ],
        out_specs=pl.BlockSpec((blk, h_out, w_out, c), lambda i: (i, 0, 0, 0)),
        compiler_params=pltpu.CompilerParams(
            dimension_semantics=("parallel",),
            vmem_limit_bytes=64 * 1024 * 1024,
        ),
        cost_estimate=pl.CostEstimate(
            flops=2 * n * c * h * w * h_out * w_out,
            transcendentals=0,
            bytes_accessed=4 * (n * c * (h * w + h_out * w_out) + h * w * h_out * w_out),
        ),
    )(k_mat, xt)

    return jnp.transpose(out_t, (0, 3, 1, 2))


def kernel(x):
    return _upsample_nhwc(x, scale=2)


# speedup vs baseline: 11.3850x; 1.0136x over previous
import functools

import numpy as np

import jax
import jax.numpy as jnp
from jax.experimental import pallas as pl
from jax.experimental.pallas import tpu as pltpu


def _interp_taps(in_size: int, out_size: int) -> np.ndarray:
    scale = np.float32(in_size / out_size)
    src = (np.arange(out_size, dtype=np.float32) + np.float32(0.5)) * scale - np.float32(0.5)
    src = np.maximum(src, np.float32(0.0))
    lo = np.minimum(np.floor(src).astype(np.int64), in_size - 1)
    hi = np.minimum(lo + 1, in_size - 1)
    frac = (src - lo.astype(np.float32)).astype(np.float32)
    mat = np.zeros((out_size, in_size), np.float32)
    np.add.at(mat, (np.arange(out_size), lo), np.float32(1.0) - frac)
    np.add.at(mat, (np.arange(out_size), hi), frac)
    return mat


def _left_kron(h: int, w: int, h_out: int, w_out: int) -> np.ndarray:
    return np.kron(_interp_taps(h, h_out), _interp_taps(w, w_out))


def _upsample_block(k_ref, x_ref, o_ref):
    b, h, w, c = x_ref.shape
    ho, wo = o_ref.shape[1], o_ref.shape[2]
    for j in range(b):
        o_ref[j] = jax.lax.dot_general(
            k_ref[...], x_ref[j].reshape(h * w, c),
            dimension_numbers=(((1,), (0,)), ((), ())),
            preferred_element_type=jnp.float32,
        ).reshape(ho, wo, c).astype(o_ref.dtype)


@functools.partial(jax.jit, static_argnames=("scale",))
def _upsample_nhwc(x: jnp.ndarray, scale: int) -> jnp.ndarray:
    n, c, h, w = x.shape
    h_out, w_out = h * scale, w * scale

    xt = jnp.transpose(x, (0, 2, 3, 1))
    k_mat = jnp.asarray(_left_kron(h, w, h_out, w_out))

    blk = 16
    while n % blk:
        blk //= 2

    out_t = pl.pallas_call(
        _upsample_block,
        out_shape=jax.ShapeDtypeStruct((n, h_out, w_out, c), x.dtype),
        grid=(n // blk,),
        in_specs=[
            pl.BlockSpec((h_out * w_out, h * w), lambda i: (0, 0)),
            pl.BlockSpec((blk, h, w, c), lambda i: (i, 0, 0, 0)),
        ],
        out_specs=pl.BlockSpec((blk, h_out, w_out, c), lambda i: (i, 0, 0, 0)),
        compiler_params=pltpu.CompilerParams(
            dimension_semantics=("parallel",),
            vmem_limit_bytes=64 * 1024 * 1024,
        ),
        cost_estimate=pl.CostEstimate(
            flops=2 * n * c * h * w * h_out * w_out,
            transcendentals=0,
            bytes_accessed=4 * (n * c * (h * w + h_out * w_out) + h * w * h_out * w_out),
        ),
    )(k_mat, xt)

    return jnp.transpose(out_t, (0, 3, 1, 2))


def kernel(x):
    return _upsample_nhwc(x, scale=2)
